# baseline (speedup 1.0000x reference)
import functools

import jax
import jax.numpy as jnp
from jax import lax
from jax.experimental import pallas as pl
from jax.experimental.pallas import tpu as pltpu
from jax.experimental.pallas import tpu_sc as plsc

_NC = 2
_NS = 16
_NW = _NC * _NS
_RB = 8


def kernel(token_ids, W):
    B, S = token_ids.shape
    dim = W.shape[1]
    flat = _gather_part(token_ids, W)
    flat2 = jnp.reshape(flat, (B, S * dim))
    return _retile(flat2, S, dim)


def _retile(y, S, dim):
    B, F = y.shape
    BB = 128

    def body(x_ref, o_ref):
        for s in range(S):
            o_ref[:, s, :] = x_ref[:, s * dim:(s + 1) * dim]

    return pl.pallas_call(
        body,
        grid=(B // BB,),
        in_specs=[pl.BlockSpec((BB, F), lambda i: (i, 0))],
        out_specs=pl.BlockSpec((BB, S, dim), lambda i: (i, 0, 0)),
        out_shape=jax.ShapeDtypeStruct((B, S, dim), y.dtype),
    )(y)


def _gather_part(token_ids, W):
    B, S = token_ids.shape
    dim = W.shape[1]
    rows_per_w = B // _NW
    rounds = rows_per_w // _RB

    mesh = plsc.VectorSubcoreMesh(core_axis_name="c", subcore_axis_name="s")

    @functools.partial(
        pl.kernel,
        mesh=mesh,
        out_type=jax.ShapeDtypeStruct((B, S, dim), W.dtype),
        scratch_types=[
            pltpu.VMEM((rows_per_w, S), jnp.int32),
            pltpu.VMEM((_RB, S, dim), jnp.float32),
            pltpu.VMEM((_RB, S, dim), jnp.float32),
            pltpu.SemaphoreType.DMA,
            pltpu.SemaphoreType.DMA,
            pltpu.SemaphoreType.DMA,
            pltpu.SemaphoreType.DMA,
        ],
        compiler_params=pltpu.CompilerParams(use_tc_tiling_on_sc=False),
    )
    def gather_kernel(w_hbm, i_hbm, o_hbm, idx_v, buf0, buf1,
                      gsem0, gsem1, wsem0, wsem1):
        wid = lax.axis_index("s") * _NC + lax.axis_index("c")
        base = wid * rows_per_w
        pltpu.sync_copy(i_hbm.at[pl.ds(base, rows_per_w)], idx_v)

        bufs = (buf0, buf1)
        gsems = (gsem0, gsem1)
        wsems = (wsem0, wsem1)

        def fire(r, slot):
            for b in range(_RB):
                pltpu.async_copy(
                    w_hbm.at[idx_v.at[r * _RB + b]],
                    bufs[slot].at[b],
                    gsems[slot],
                )

        def drain_gathers(slot):
            pltpu.make_async_copy(
                o_hbm.at[pl.ds(0, _RB)], bufs[slot], gsems[slot]
            ).wait()

        def start_wb(r, slot):
            pltpu.async_copy(
                bufs[slot], o_hbm.at[pl.ds(base + r * _RB, _RB)], wsems[slot]
            )

        def drain_wb(slot):
            pltpu.make_async_copy(
                bufs[slot], o_hbm.at[pl.ds(base, _RB)], wsems[slot]
            ).wait()

        fire(0, 0)
        drain_gathers(0)
        fire(1, 1)
        start_wb(0, 0)
        drain_gathers(1)
        drain_wb(0)
        fire(2, 0)
        start_wb(1, 1)

        @pl.loop(2, rounds - 2, step=2)
        def _(j):
            drain_gathers(0)
            drain_wb(1)
            fire(j + 1, 1)
            start_wb(j, 0)
            drain_gathers(1)
            drain_wb(0)
            fire(j + 2, 0)
            start_wb(j + 1, 1)

        drain_gathers(0)
        drain_wb(1)
        fire(rounds - 1, 1)
        start_wb(rounds - 2, 0)
        drain_gathers(1)
        drain_wb(0)
        start_wb(rounds - 1, 1)
        drain_wb(1)

    return gather_kernel(W, token_ids)


# speedup vs baseline: 1.1839x; 1.1839x over previous
import functools

import jax
import jax.numpy as jnp
from jax import lax
from jax.experimental import pallas as pl
from jax.experimental.pallas import tpu as pltpu
from jax.experimental.pallas import tpu_sc as plsc

_NC = 2
_NS = 16
_NW = _NC * _NS
_RB = 8


def kernel(token_ids, W):
    B, S = token_ids.shape
    dim = W.shape[1]
    flat = _gather_part(token_ids, W)
    flat2 = jnp.reshape(flat, (B, S * dim))
    return _retile(flat2, S, dim)


def _retile(y, S, dim):
    B, F = y.shape
    BB = 128

    def body(x_ref, o_ref):
        o_ref[...] = x_ref[...].reshape(BB, S, dim)

    return pl.pallas_call(
        body,
        grid=(B // BB,),
        in_specs=[pl.BlockSpec((BB, F), lambda i: (i, 0))],
        out_specs=pl.BlockSpec((BB, S, dim), lambda i: (i, 0, 0)),
        out_shape=jax.ShapeDtypeStruct((B, S, dim), y.dtype),
    )(y)


def _gather_part(token_ids, W):
    B, S = token_ids.shape
    dim = W.shape[1]
    rows_per_w = B // _NW
    rounds = rows_per_w // _RB

    mesh = plsc.VectorSubcoreMesh(core_axis_name="c", subcore_axis_name="s")

    @functools.partial(
        pl.kernel,
        mesh=mesh,
        out_type=jax.ShapeDtypeStruct((B, S, dim), W.dtype),
        scratch_types=[
            pltpu.VMEM((rows_per_w, S), jnp.int32),
            pltpu.VMEM((_RB, S, dim), jnp.float32),
            pltpu.VMEM((_RB, S, dim), jnp.float32),
            pltpu.SemaphoreType.DMA,
            pltpu.SemaphoreType.DMA,
            pltpu.SemaphoreType.DMA,
            pltpu.SemaphoreType.DMA,
        ],
        compiler_params=pltpu.CompilerParams(use_tc_tiling_on_sc=False),
    )
    def gather_kernel(w_hbm, i_hbm, o_hbm, idx_v, buf0, buf1,
                      gsem0, gsem1, wsem0, wsem1):
        wid = lax.axis_index("s") * _NC + lax.axis_index("c")
        base = wid * rows_per_w
        pltpu.sync_copy(i_hbm.at[pl.ds(base, rows_per_w)], idx_v)

        bufs = (buf0, buf1)
        gsems = (gsem0, gsem1)
        wsems = (wsem0, wsem1)

        def fire(r, slot):
            for b in range(_RB):
                pltpu.async_copy(
                    w_hbm.at[idx_v.at[r * _RB + b]],
                    bufs[slot].at[b],
                    gsems[slot],
                )

        def drain_gathers(slot):
            pltpu.make_async_copy(
                o_hbm.at[pl.ds(0, _RB)], bufs[slot], gsems[slot]
            ).wait()

        def start_wb(r, slot):
            pltpu.async_copy(
                bufs[slot], o_hbm.at[pl.ds(base + r * _RB, _RB)], wsems[slot]
            )

        def drain_wb(slot):
            pltpu.make_async_copy(
                bufs[slot], o_hbm.at[pl.ds(base, _RB)], wsems[slot]
            ).wait()

        fire(0, 0)
        drain_gathers(0)
        fire(1, 1)
        start_wb(0, 0)
        drain_gathers(1)
        drain_wb(0)
        fire(2, 0)
        start_wb(1, 1)

        @pl.loop(2, rounds - 2, step=2)
        def _(j):
            drain_gathers(0)
            drain_wb(1)
            fire(j + 1, 1)
            start_wb(j, 0)
            drain_gathers(1)
            drain_wb(0)
            fire(j + 2, 0)
            start_wb(j + 1, 1)

        drain_gathers(0)
        drain_wb(1)
        fire(rounds - 1, 1)
        start_wb(rounds - 2, 0)
        drain_gathers(1)
        drain_wb(0)
        start_wb(rounds - 1, 1)
        drain_wb(1)

    return gather_kernel(W, token_ids)


# speedup vs baseline: 3.0770x; 2.5991x over previous
import functools

import jax
import jax.numpy as jnp
from jax import lax
from jax.experimental import pallas as pl
from jax.experimental.pallas import tpu as pltpu
from jax.experimental.pallas import tpu_sc as plsc

_NC = 2
_NS = 16
_NW = _NC * _NS
_RB = 8
_PS = 56
_PD = 128


def kernel(token_ids, W):
    B, S = token_ids.shape
    dim = W.shape[1]
    padded = _gather_padded(token_ids, W)
    return padded[:, :S, :dim]


def _gather_padded(token_ids, W):
    B, S = token_ids.shape
    dim = W.shape[1]
    rows_per_w = B // _NW
    rounds = rows_per_w // _RB

    mesh = plsc.VectorSubcoreMesh(core_axis_name="c", subcore_axis_name="s")

    @functools.partial(
        pl.kernel,
        mesh=mesh,
        out_type=jax.ShapeDtypeStruct((B, _PS, _PD), W.dtype),
        scratch_types=[
            pltpu.VMEM((rows_per_w, S), jnp.int32),
            pltpu.VMEM((_RB, S, dim), jnp.float32),
            pltpu.VMEM((_RB, S, dim), jnp.float32),
            pltpu.SemaphoreType.DMA,
            pltpu.SemaphoreType.DMA,
            pltpu.SemaphoreType.DMA,
            pltpu.SemaphoreType.DMA,
        ],
        compiler_params=pltpu.CompilerParams(use_tc_tiling_on_sc=False),
    )
    def gather_kernel(w_hbm, i_hbm, o_hbm, idx_v, buf0, buf1,
                      gsem0, gsem1, wsem0, wsem1):
        wid = lax.axis_index("s") * _NC + lax.axis_index("c")
        base = wid * rows_per_w
        pltpu.sync_copy(i_hbm.at[pl.ds(base, rows_per_w)], idx_v)

        bufs = (buf0, buf1)
        gsems = (gsem0, gsem1)
        wsems = (wsem0, wsem1)

        def fire(r, slot):
            for b in range(_RB):
                pltpu.async_copy(
                    w_hbm.at[idx_v.at[r * _RB + b]],
                    bufs[slot].at[b],
                    gsems[slot],
                )

        def drain_gathers(slot):
            pltpu.make_async_copy(
                o_hbm.at[pl.ds(0, _RB), pl.ds(0, S), pl.ds(0, dim)],
                bufs[slot],
                gsems[slot],
            ).wait()

        def start_wb(r, slot):
            for b in range(_RB):
                pltpu.async_copy(
                    bufs[slot].at[b],
                    o_hbm.at[base + r * _RB + b, pl.ds(0, S), pl.ds(0, dim)],
                    wsems[slot],
                )

        def drain_wb(slot):
            pltpu.make_async_copy(
                bufs[slot],
                o_hbm.at[pl.ds(0, _RB), pl.ds(0, S), pl.ds(0, dim)],
                wsems[slot],
            ).wait()

        fire(0, 0)
        drain_gathers(0)
        fire(1, 1)
        start_wb(0, 0)
        drain_gathers(1)
        drain_wb(0)
        fire(2, 0)
        start_wb(1, 1)

        @pl.loop(2, rounds - 2, step=2)
        def _(j):
            drain_gathers(0)
            drain_wb(1)
            fire(j + 1, 1)
            start_wb(j, 0)
            drain_gathers(1)
            drain_wb(0)
            fire(j + 2, 0)
            start_wb(j + 1, 1)

        drain_gathers(0)
        drain_wb(1)
        fire(rounds - 1, 1)
        start_wb(rounds - 2, 0)
        drain_gathers(1)
        drain_wb(0)
        start_wb(rounds - 1, 1)
        drain_wb(1)

    return gather_kernel(W, token_ids)


# speedup vs baseline: 3.0777x; 1.0002x over previous
import functools

import jax
import jax.numpy as jnp
from jax import lax
from jax.experimental import pallas as pl
from jax.experimental.pallas import tpu as pltpu
from jax.experimental.pallas import tpu_sc as plsc

_NC = 2
_NS = 16
_NW = _NC * _NS
_RB = 8
_PS = 56
_PD = 128


def kernel(token_ids, W):
    B, S = token_ids.shape
    dim = W.shape[1]
    padded = _gather_padded(token_ids, W)
    return padded[:, :S, :dim]


def _gather_padded(token_ids, W):
    B, S = token_ids.shape
    dim = W.shape[1]
    rows_per_w = B // _NW
    rounds = rows_per_w // _RB

    mesh = plsc.VectorSubcoreMesh(core_axis_name="c", subcore_axis_name="s")

    @functools.partial(
        pl.kernel,
        mesh=mesh,
        out_type=jax.ShapeDtypeStruct((B, _PS, _PD), W.dtype),
        scratch_types=[
            pltpu.VMEM((rows_per_w * S // 100, 100), jnp.int32),
            pltpu.VMEM((_RB * S, dim), jnp.float32),
            pltpu.VMEM((_RB * S, dim), jnp.float32),
            pltpu.SemaphoreType.DMA,
            pltpu.SemaphoreType.DMA,
            pltpu.SemaphoreType.DMA,
            pltpu.SemaphoreType.DMA,
        ],
        compiler_params=pltpu.CompilerParams(use_tc_tiling_on_sc=False),
    )
    def gather_kernel(w_hbm, i_hbm, o_hbm, idx_v, buf0, buf1,
                      gsem0, gsem1, wsem0, wsem1):
        wid = lax.axis_index("s") * _NC + lax.axis_index("c")
        base = wid * rows_per_w
        idx_rows = rows_per_w * S // 100
        pltpu.sync_copy(i_hbm.at[pl.ds(wid * idx_rows, idx_rows)], idx_v)

        bufs = (buf0, buf1)
        gsems = (gsem0, gsem1)
        wsems = (wsem0, wsem1)

        streams_per_round = _RB * S // 100

        def fire(r, slot):
            for k in range(streams_per_round):
                pltpu.async_copy(
                    w_hbm.at[idx_v.at[r * streams_per_round + k]],
                    bufs[slot].at[pl.ds(k * 100, 100)],
                    gsems[slot],
                )

        def drain_gathers(slot):
            pltpu.make_async_copy(
                w_hbm.at[pl.ds(0, _RB * S)],
                bufs[slot],
                gsems[slot],
            ).wait()

        def start_wb(r, slot):
            for b in range(_RB):
                pltpu.async_copy(
                    bufs[slot].at[pl.ds(b * S, S)],
                    o_hbm.at[base + r * _RB + b, pl.ds(0, S), pl.ds(0, dim)],
                    wsems[slot],
                )

        def drain_wb(slot):
            pltpu.make_async_copy(
                w_hbm.at[pl.ds(0, _RB * S)],
                bufs[slot],
                wsems[slot],
            ).wait()

        fire(0, 0)
        drain_gathers(0)
        fire(1, 1)
        start_wb(0, 0)
        drain_gathers(1)
        drain_wb(0)
        fire(2, 0)
        start_wb(1, 1)

        @pl.loop(2, rounds - 2, step=2)
        def _(j):
            drain_gathers(0)
            drain_wb(1)
            fire(j + 1, 1)
            start_wb(j, 0)
            drain_gathers(1)
            drain_wb(0)
            fire(j + 2, 0)
            start_wb(j + 1, 1)

        drain_gathers(0)
        drain_wb(1)
        fire(rounds - 1, 1)
        start_wb(rounds - 2, 0)
        drain_gathers(1)
        drain_wb(0)
        start_wb(rounds - 1, 1)
        drain_wb(1)

    return gather_kernel(W, token_ids.reshape(B * S // 100, 100))
